# baseline (speedup 1.0000x reference)
import functools

import jax
import jax.numpy as jnp
from jax import lax
from jax.experimental import pallas as pl
from jax.experimental.pallas import tpu as pltpu
from jax.experimental.pallas import tpu_sc as plsc

N = 10000
E = 320000
D = 128

NC = 2
NS = 16
NW = NC * NS
CH = 80
N_PAD = 10240
RPW = N_PAD // NS
EPW = E // NW
NCHUNK = EPW // CH
BLK = 512

_mesh = plsc.VectorSubcoreMesh(core_axis_name="c", subcore_axis_name="s",
                               num_cores=NC, num_subcores=NS)


@functools.partial(
    pl.kernel,
    out_type=jax.ShapeDtypeStruct((NC, N_PAD, D), jnp.float32),
    mesh=_mesh,
    scratch_types=[
        pltpu.VMEM_SHARED((N_PAD, D), jnp.float32),
        pltpu.VMEM((CH,), jnp.int32),
        pltpu.VMEM((CH,), jnp.int32),
        pltpu.VMEM((CH, D), jnp.float32),
        pltpu.SemaphoreType.DMA,
        pltpu.SemaphoreType.DMA,
    ],
)
def _deg_kernel(dst_hbm, ones_hbm, zeros_hbm, out_hbm, acc_sh,
                dst0, dst1, ones_v, dsem0, dsem1):
    cid = lax.axis_index("c")
    sid = lax.axis_index("s")
    wid = cid * NS + sid
    ebase = wid * EPW
    pltpu.sync_copy(ones_hbm, ones_v)
    pltpu.sync_copy(zeros_hbm.at[pl.ds(sid * RPW, RPW)],
                    acc_sh.at[pl.ds(sid * RPW, RPW)])
    plsc.subcore_barrier()

    def _dload(i, dstb, sem):
        pltpu.async_copy(dst_hbm.at[pl.ds(ebase + i * CH, CH)], dstb, sem)

    def _dwait(dstb, sem):
        pltpu.make_async_copy(dst_hbm.at[pl.ds(0, CH)], dstb, sem).wait()

    _dload(0, dst0, dsem0)

    def step2(t, carry):
        i0 = 2 * t
        _dload(i0 + 1, dst1, dsem1)
        _dwait(dst0, dsem0)
        pltpu.sync_copy(ones_v, acc_sh.at[dst0], add=True)
        _dload(i0 + 2, dst0, dsem0)
        _dwait(dst1, dsem1)
        pltpu.sync_copy(ones_v, acc_sh.at[dst1], add=True)
        return carry

    lax.fori_loop(0, NCHUNK // 2, step2, 0)
    _dwait(dst0, dsem0)
    pltpu.sync_copy(ones_v, acc_sh.at[dst0], add=True)
    plsc.subcore_barrier()
    pltpu.sync_copy(acc_sh.at[pl.ds(sid * RPW, RPW)],
                    out_hbm.at[cid, pl.ds(sid * RPW, RPW)])


@functools.partial(
    pl.kernel,
    out_type=jax.ShapeDtypeStruct((NC, N_PAD, D), jnp.float32),
    mesh=_mesh,
    scratch_types=[
        pltpu.VMEM_SHARED((N_PAD, D), jnp.float32),
        pltpu.VMEM((EPW,), jnp.int32),
        pltpu.VMEM((CH,), jnp.int32),
        pltpu.VMEM((CH,), jnp.int32),
        pltpu.VMEM((CH, D), jnp.float32),
        pltpu.VMEM((CH, D), jnp.float32),
        pltpu.SemaphoreType.DMA,
        pltpu.SemaphoreType.DMA,
        pltpu.SemaphoreType.DMA,
        pltpu.SemaphoreType.DMA,
    ],
)
def _edge_kernel(g_hbm, src_hbm, dst_hbm, zeros_hbm, out_hbm,
                 acc_sh, src_all, dst0, dst1, rows0, rows1,
                 gsem0, gsem1, dsem0, dsem1):
    cid = lax.axis_index("c")
    sid = lax.axis_index("s")
    wid = cid * NS + sid
    ebase = wid * EPW
    pltpu.sync_copy(src_hbm.at[pl.ds(ebase, EPW)], src_all)
    pltpu.sync_copy(zeros_hbm.at[pl.ds(sid * RPW, RPW)],
                    acc_sh.at[pl.ds(sid * RPW, RPW)])
    plsc.subcore_barrier()

    def _gather(i, rows, sem):
        pltpu.async_copy(g_hbm.at[src_all.at[pl.ds(i * CH, CH)]], rows, sem)

    def _dload(i, dstb, sem):
        pltpu.async_copy(dst_hbm.at[pl.ds(ebase + i * CH, CH)], dstb, sem)

    def _wait(dst_buf, sem):
        pltpu.make_async_copy(zeros_hbm.at[pl.ds(0, CH)]
                              if dst_buf.shape == (CH, D) else
                              dst_hbm.at[pl.ds(0, CH)],
                              dst_buf, sem).wait()

    _gather(0, rows0, gsem0)
    _dload(0, dst0, dsem0)

    def step2(t, carry):
        i0 = 2 * t
        _gather(i0 + 1, rows1, gsem1)
        _dload(i0 + 1, dst1, dsem1)
        _wait(rows0, gsem0)
        _wait(dst0, dsem0)
        pltpu.sync_copy(rows0, acc_sh.at[dst0], add=True)
        _gather(i0 + 2, rows0, gsem0)
        _dload(i0 + 2, dst0, dsem0)
        _wait(rows1, gsem1)
        _wait(dst1, dsem1)
        pltpu.sync_copy(rows1, acc_sh.at[dst1], add=True)
        return carry

    lax.fori_loop(0, NCHUNK // 2, step2, 0)
    _wait(rows0, gsem0)
    _wait(dst0, dsem0)
    pltpu.sync_copy(rows0, acc_sh.at[dst0], add=True)
    plsc.subcore_barrier()
    pltpu.sync_copy(acc_sh.at[pl.ds(sid * RPW, RPW)],
                    out_hbm.at[cid, pl.ds(sid * RPW, RPW)])


def _t_matmul_body(x_ref, w_ref, h_ref):
    h_ref[...] = jnp.dot(x_ref[...], w_ref[...],
                         preferred_element_type=jnp.float32)


def _t_scale_body(h_ref, q_ref, g_ref, disv_ref):
    deg = q_ref[0, :, 0:1] + q_ref[1, :, 0:1] + 1.0
    dis = lax.rsqrt(deg)
    g_ref[...] = h_ref[...] * dis
    disv_ref[...] = jnp.broadcast_to(dis, (BLK, 8))


def _t_mid_body(p_ref, g_ref, disv_ref, b_ref, w_ref, gout_ref):
    dis = disv_ref[:, 0:1]
    p = p_ref[...]
    y = dis * (p[0] + p[1] + g_ref[...]) + b_ref[...]
    y = jnp.maximum(y, 0.0)
    h = jnp.dot(y, w_ref[...], preferred_element_type=jnp.float32)
    gout_ref[...] = h * dis


def _t_last_body(p_ref, g_ref, disv_ref, b_ref, out_ref):
    dis = disv_ref[:, 0:1]
    p = p_ref[...]
    out_ref[...] = dis * (p[0] + p[1] + g_ref[...]) + b_ref[...]


_GRID = (N_PAD // BLK,)
_spec_rows = pl.BlockSpec((BLK, D), lambda i: (i, 0))
_spec_p = pl.BlockSpec((2, BLK, D), lambda i: (0, i, 0))
_spec_q = pl.BlockSpec((2, BLK, D), lambda i: (0, i, 0))
_spec_dis = pl.BlockSpec((BLK, 8), lambda i: (i, 0))
_spec_w = pl.BlockSpec((D, D), lambda i: (0, 0))
_spec_b = pl.BlockSpec((1, D), lambda i: (0, 0))
_out_rows = jax.ShapeDtypeStruct((N_PAD, D), jnp.float32)
_out_dis = jax.ShapeDtypeStruct((N_PAD, 8), jnp.float32)

_t_matmul = pl.pallas_call(
    _t_matmul_body, grid=_GRID,
    in_specs=[_spec_rows, _spec_w],
    out_specs=_spec_rows, out_shape=_out_rows)

_t_scale = pl.pallas_call(
    _t_scale_body, grid=_GRID,
    in_specs=[_spec_rows, _spec_q],
    out_specs=(_spec_rows, _spec_dis), out_shape=(_out_rows, _out_dis))

_t_mid = pl.pallas_call(
    _t_mid_body, grid=_GRID,
    in_specs=[_spec_p, _spec_rows, _spec_dis, _spec_b, _spec_w],
    out_specs=_spec_rows, out_shape=_out_rows)

_t_last = pl.pallas_call(
    _t_last_body, grid=_GRID,
    in_specs=[_spec_p, _spec_rows, _spec_dis, _spec_b],
    out_specs=_spec_rows, out_shape=_out_rows)


def kernel(x, edge_index, W1, b1, W2, b2, W3, b3):
    src_p = edge_index[0].astype(jnp.int32)
    dst_p = edge_index[1].astype(jnp.int32)
    x_p = jnp.pad(x, ((0, N_PAD - N), (0, 0)))
    zeros_rows = jnp.zeros((N_PAD, D), jnp.float32)
    ones_rows = jnp.ones((CH, D), jnp.float32)
    b1r = b1.reshape(1, D)
    b2r = b2.reshape(1, D)
    b3r = b3.reshape(1, D)

    q = _deg_kernel(dst_p, ones_rows, zeros_rows)
    h1 = _t_matmul(x_p, W1)
    g1, disv = _t_scale(h1, q)
    p1 = _edge_kernel(g1, src_p, dst_p, zeros_rows)
    g2 = _t_mid(p1, g1, disv, b1r, W2)
    p2 = _edge_kernel(g2, src_p, dst_p, zeros_rows)
    g3 = _t_mid(p2, g2, disv, b2r, W3)
    p3 = _edge_kernel(g3, src_p, dst_p, zeros_rows)
    out = _t_last(p3, g3, disv, b3r)
    return out[:N]


# speedup vs baseline: 1.1715x; 1.1715x over previous
import functools

import jax
import jax.numpy as jnp
from jax import lax
from jax.experimental import pallas as pl
from jax.experimental.pallas import tpu as pltpu
from jax.experimental.pallas import tpu_sc as plsc

N = 10000
E = 320000
D = 128

NC = 2
NS = 16
NW = NC * NS
CH = 128
N_PAD = 10240
RPW = N_PAD // NS
EPW = E // NW
NCHUNK = EPW // CH
TAIL = EPW - NCHUNK * CH
BLK = 1280
BLKL = 1000

_mesh = plsc.VectorSubcoreMesh(core_axis_name="c", subcore_axis_name="s",
                               num_cores=NC, num_subcores=NS)


@functools.partial(
    pl.kernel,
    out_type=jax.ShapeDtypeStruct((NC, N_PAD, D), jnp.float32),
    mesh=_mesh,
    scratch_types=[
        pltpu.VMEM_SHARED((N_PAD, D), jnp.float32),
        pltpu.VMEM((CH,), jnp.int32),
        pltpu.VMEM((CH,), jnp.int32),
        pltpu.VMEM((TAIL,), jnp.int32),
        pltpu.VMEM((CH, D), jnp.float32),
        pltpu.SemaphoreType.DMA,
        pltpu.SemaphoreType.DMA,
        pltpu.SemaphoreType.DMA,
    ],
)
def _deg_kernel(dst_hbm, ones_hbm, zeros_hbm, out_hbm, acc_sh,
                dst0, dst1, dstt, ones_v, dsem0, dsem1, tsem):
    cid = lax.axis_index("c")
    sid = lax.axis_index("s")
    wid = cid * NS + sid
    ebase = wid * EPW
    pltpu.sync_copy(ones_hbm, ones_v)
    pltpu.sync_copy(zeros_hbm.at[pl.ds(sid * RPW, RPW)],
                    acc_sh.at[pl.ds(sid * RPW, RPW)])
    plsc.subcore_barrier()

    def _dload(i, dstb, sem):
        pltpu.async_copy(dst_hbm.at[pl.ds(ebase + i * CH, CH)], dstb, sem)

    def _dwait(dstb, sem):
        pltpu.make_async_copy(dst_hbm.at[pl.ds(0, CH)], dstb, sem).wait()

    _dload(0, dst0, dsem0)
    pltpu.async_copy(dst_hbm.at[pl.ds(ebase + NCHUNK * CH, TAIL)], dstt, tsem)

    def step2(t, carry):
        i0 = 2 * t
        _dload(i0 + 1, dst1, dsem1)
        _dwait(dst0, dsem0)
        pltpu.sync_copy(ones_v, acc_sh.at[dst0], add=True)

        @pl.when(t < NCHUNK // 2 - 1)
        def _():
            _dload(i0 + 2, dst0, dsem0)

        _dwait(dst1, dsem1)
        pltpu.sync_copy(ones_v, acc_sh.at[dst1], add=True)
        return carry

    lax.fori_loop(0, NCHUNK // 2, step2, 0)
    pltpu.make_async_copy(dst_hbm.at[pl.ds(0, TAIL)], dstt, tsem).wait()
    pltpu.sync_copy(ones_v.at[pl.ds(0, TAIL)], acc_sh.at[dstt], add=True)
    plsc.subcore_barrier()
    pltpu.sync_copy(acc_sh.at[pl.ds(sid * RPW, RPW)],
                    out_hbm.at[cid, pl.ds(sid * RPW, RPW)])


@functools.partial(
    pl.kernel,
    out_type=jax.ShapeDtypeStruct((NC, N_PAD, D), jnp.float32),
    mesh=_mesh,
    scratch_types=[
        pltpu.VMEM_SHARED((N_PAD, D), jnp.float32),
        pltpu.VMEM((EPW,), jnp.int32),
        pltpu.VMEM((CH,), jnp.int32),
        pltpu.VMEM((CH,), jnp.int32),
        pltpu.VMEM((TAIL,), jnp.int32),
        pltpu.VMEM((CH, D), jnp.float32),
        pltpu.VMEM((CH, D), jnp.float32),
        pltpu.VMEM((TAIL, D), jnp.float32),
        pltpu.SemaphoreType.DMA,
        pltpu.SemaphoreType.DMA,
        pltpu.SemaphoreType.DMA,
        pltpu.SemaphoreType.DMA,
        pltpu.SemaphoreType.DMA,
    ],
)
def _edge_kernel(g_hbm, src_hbm, dst_hbm, zeros_hbm, out_hbm,
                 acc_sh, src_all, dst0, dst1, dstt, rows0, rows1, rowst,
                 gsem0, gsem1, dsem0, dsem1, tsem):
    cid = lax.axis_index("c")
    sid = lax.axis_index("s")
    wid = cid * NS + sid
    ebase = wid * EPW
    pltpu.sync_copy(src_hbm.at[pl.ds(ebase, EPW)], src_all)
    pltpu.sync_copy(zeros_hbm.at[pl.ds(sid * RPW, RPW)],
                    acc_sh.at[pl.ds(sid * RPW, RPW)])
    plsc.subcore_barrier()

    def _gather(i, rows, sem):
        pltpu.async_copy(g_hbm.at[src_all.at[pl.ds(i * CH, CH)]], rows, sem)

    def _dload(i, dstb, sem):
        pltpu.async_copy(dst_hbm.at[pl.ds(ebase + i * CH, CH)], dstb, sem)

    def _wait(dst_buf, sem, n):
        pltpu.make_async_copy(zeros_hbm.at[pl.ds(0, n)]
                              if len(dst_buf.shape) == 2 else
                              dst_hbm.at[pl.ds(0, n)],
                              dst_buf, sem).wait()

    _gather(0, rows0, gsem0)
    _dload(0, dst0, dsem0)
    pltpu.async_copy(
        g_hbm.at[src_all.at[pl.ds(NCHUNK * CH, TAIL)]], rowst, tsem)
    pltpu.async_copy(dst_hbm.at[pl.ds(ebase + NCHUNK * CH, TAIL)], dstt, tsem)

    def step2(t, carry):
        i0 = 2 * t
        _gather(i0 + 1, rows1, gsem1)
        _dload(i0 + 1, dst1, dsem1)
        _wait(rows0, gsem0, CH)
        _wait(dst0, dsem0, CH)
        pltpu.sync_copy(rows0, acc_sh.at[dst0], add=True)

        @pl.when(t < NCHUNK // 2 - 1)
        def _():
            _gather(i0 + 2, rows0, gsem0)
            _dload(i0 + 2, dst0, dsem0)

        _wait(rows1, gsem1, CH)
        _wait(dst1, dsem1, CH)
        pltpu.sync_copy(rows1, acc_sh.at[dst1], add=True)
        return carry

    lax.fori_loop(0, NCHUNK // 2, step2, 0)
    _wait(rowst, tsem, TAIL)
    _wait(dstt, tsem, TAIL)
    pltpu.sync_copy(rowst, acc_sh.at[dstt], add=True)
    plsc.subcore_barrier()
    pltpu.sync_copy(acc_sh.at[pl.ds(sid * RPW, RPW)],
                    out_hbm.at[cid, pl.ds(sid * RPW, RPW)])


def _t_matmul_body(x_ref, w_ref, h_ref):
    h_ref[...] = jnp.dot(x_ref[...], w_ref[...],
                         preferred_element_type=jnp.float32)


def _t_scale_body(h_ref, q_ref, g_ref, disv_ref):
    deg = q_ref[0, :, 0:1] + q_ref[1, :, 0:1] + 1.0
    dis = lax.rsqrt(deg)
    g_ref[...] = h_ref[...] * dis
    disv_ref[...] = jnp.broadcast_to(dis, (BLK, 8))


def _t_mid_body(p_ref, g_ref, disv_ref, b_ref, w_ref, gout_ref):
    dis = disv_ref[:, 0:1]
    p = p_ref[...]
    y = dis * (p[0] + p[1] + g_ref[...]) + b_ref[...]
    y = jnp.maximum(y, 0.0)
    h = jnp.dot(y, w_ref[...], preferred_element_type=jnp.float32)
    gout_ref[...] = h * dis


def _t_last_body(p_ref, g_ref, disv_ref, b_ref, out_ref):
    dis = disv_ref[:, 0:1]
    p = p_ref[...]
    out_ref[...] = dis * (p[0] + p[1] + g_ref[...]) + b_ref[...]


def _t_split_body(e_ref, s_ref, d_ref):
    s_ref[...] = e_ref[0]
    d_ref[...] = e_ref[1]


_out_e = jax.ShapeDtypeStruct((E,), jnp.int32)
_t_split = pl.pallas_call(
    _t_split_body,
    in_specs=[pl.BlockSpec((2, E), lambda: (0, 0))],
    out_specs=(pl.BlockSpec((E,), lambda: (0,)),
               pl.BlockSpec((E,), lambda: (0,))),
    out_shape=(_out_e, _out_e))

_GRID = (N_PAD // BLK,)
_spec_rows = pl.BlockSpec((BLK, D), lambda i: (i, 0))
_spec_p = pl.BlockSpec((2, BLK, D), lambda i: (0, i, 0))
_spec_q = pl.BlockSpec((2, BLK, D), lambda i: (0, i, 0))
_spec_dis = pl.BlockSpec((BLK, 8), lambda i: (i, 0))
_spec_w = pl.BlockSpec((D, D), lambda i: (0, 0))
_spec_b = pl.BlockSpec((1, D), lambda i: (0, 0))
_out_rows = jax.ShapeDtypeStruct((N_PAD, D), jnp.float32)
_out_dis = jax.ShapeDtypeStruct((N_PAD, 8), jnp.float32)

_t_matmul = pl.pallas_call(
    _t_matmul_body, grid=_GRID,
    in_specs=[_spec_rows, _spec_w],
    out_specs=_spec_rows, out_shape=_out_rows)

_t_scale = pl.pallas_call(
    _t_scale_body, grid=_GRID,
    in_specs=[_spec_rows, _spec_q],
    out_specs=(_spec_rows, _spec_dis), out_shape=(_out_rows, _out_dis))

_t_mid = pl.pallas_call(
    _t_mid_body, grid=_GRID,
    in_specs=[_spec_p, _spec_rows, _spec_dis, _spec_b, _spec_w],
    out_specs=_spec_rows, out_shape=_out_rows)

_t_last = pl.pallas_call(
    _t_last_body, grid=(N // BLKL,),
    in_specs=[pl.BlockSpec((2, BLKL, D), lambda i: (0, i, 0)),
              pl.BlockSpec((BLKL, D), lambda i: (i, 0)),
              pl.BlockSpec((BLKL, 8), lambda i: (i, 0)),
              _spec_b],
    out_specs=pl.BlockSpec((BLKL, D), lambda i: (i, 0)),
    out_shape=jax.ShapeDtypeStruct((N, D), jnp.float32))


def kernel(x, edge_index, W1, b1, W2, b2, W3, b3):
    src_p, dst_p = _t_split(edge_index.astype(jnp.int32))
    x_p = jnp.pad(x, ((0, N_PAD - N), (0, 0)))
    zeros_rows = jnp.zeros((N_PAD, D), jnp.float32)
    ones_rows = jnp.ones((CH, D), jnp.float32)
    b1r = b1.reshape(1, D)
    b2r = b2.reshape(1, D)
    b3r = b3.reshape(1, D)

    q = _deg_kernel(dst_p, ones_rows, zeros_rows)
    h1 = _t_matmul(x_p, W1)
    g1, disv = _t_scale(h1, q)
    p1 = _edge_kernel(g1, src_p, dst_p, zeros_rows)
    g2 = _t_mid(p1, g1, disv, b1r, W2)
    p2 = _edge_kernel(g2, src_p, dst_p, zeros_rows)
    g3 = _t_mid(p2, g2, disv, b2r, W3)
    p3 = _edge_kernel(g3, src_p, dst_p, zeros_rows)
    return _t_last(p3, g3, disv, b3r)


# speedup vs baseline: 1.1791x; 1.0065x over previous
import functools

import jax
import jax.numpy as jnp
from jax import lax
from jax.experimental import pallas as pl
from jax.experimental.pallas import tpu as pltpu
from jax.experimental.pallas import tpu_sc as plsc

N = 10000
E = 320000
D = 128

NC = 2
NS = 16
NW = NC * NS
CH = 128
N_PAD = 10240
RPW = N_PAD // NS
EPW = E // NW
NCHUNK = EPW // CH
TAIL = EPW - NCHUNK * CH
BLK = 2560
BLKL = 1000

_mesh = plsc.VectorSubcoreMesh(core_axis_name="c", subcore_axis_name="s",
                               num_cores=NC, num_subcores=NS)


@functools.partial(
    pl.kernel,
    out_type=jax.ShapeDtypeStruct((NC, N_PAD, D), jnp.float32),
    mesh=_mesh,
    scratch_types=[
        pltpu.VMEM_SHARED((N_PAD, D), jnp.float32),
        pltpu.VMEM((CH,), jnp.int32),
        pltpu.VMEM((CH,), jnp.int32),
        pltpu.VMEM((TAIL,), jnp.int32),
        pltpu.VMEM((CH, D), jnp.float32),
        pltpu.SemaphoreType.DMA,
        pltpu.SemaphoreType.DMA,
        pltpu.SemaphoreType.DMA,
    ],
)
def _deg_kernel(dst_hbm, ones_hbm, zeros_hbm, out_hbm, acc_sh,
                dst0, dst1, dstt, ones_v, dsem0, dsem1, tsem):
    cid = lax.axis_index("c")
    sid = lax.axis_index("s")
    wid = cid * NS + sid
    ebase = wid * EPW
    pltpu.sync_copy(ones_hbm, ones_v)
    pltpu.sync_copy(zeros_hbm.at[pl.ds(sid * RPW, RPW)],
                    acc_sh.at[pl.ds(sid * RPW, RPW)])
    plsc.subcore_barrier()

    def _dload(i, dstb, sem):
        pltpu.async_copy(dst_hbm.at[pl.ds(ebase + i * CH, CH)], dstb, sem)

    def _dwait(dstb, sem):
        pltpu.make_async_copy(dst_hbm.at[pl.ds(0, CH)], dstb, sem).wait()

    _dload(0, dst0, dsem0)
    pltpu.async_copy(dst_hbm.at[pl.ds(ebase + NCHUNK * CH, TAIL)], dstt, tsem)

    def step2(t, carry):
        i0 = 2 * t
        _dload(i0 + 1, dst1, dsem1)
        _dwait(dst0, dsem0)
        pltpu.sync_copy(ones_v, acc_sh.at[dst0], add=True)

        @pl.when(t < NCHUNK // 2 - 1)
        def _():
            _dload(i0 + 2, dst0, dsem0)

        _dwait(dst1, dsem1)
        pltpu.sync_copy(ones_v, acc_sh.at[dst1], add=True)
        return carry

    lax.fori_loop(0, NCHUNK // 2, step2, 0)
    pltpu.make_async_copy(dst_hbm.at[pl.ds(0, TAIL)], dstt, tsem).wait()
    pltpu.sync_copy(ones_v.at[pl.ds(0, TAIL)], acc_sh.at[dstt], add=True)
    plsc.subcore_barrier()
    pltpu.sync_copy(acc_sh.at[pl.ds(sid * RPW, RPW)],
                    out_hbm.at[cid, pl.ds(sid * RPW, RPW)])


@functools.partial(
    pl.kernel,
    out_type=jax.ShapeDtypeStruct((NC, N_PAD, D), jnp.float32),
    mesh=_mesh,
    scratch_types=[
        pltpu.VMEM_SHARED((N_PAD, D), jnp.float32),
        pltpu.VMEM((EPW,), jnp.int32),
        pltpu.VMEM((CH,), jnp.int32),
        pltpu.VMEM((CH,), jnp.int32),
        pltpu.VMEM((TAIL,), jnp.int32),
        pltpu.VMEM((CH, D), jnp.float32),
        pltpu.VMEM((CH, D), jnp.float32),
        pltpu.VMEM((TAIL, D), jnp.float32),
        pltpu.SemaphoreType.DMA,
        pltpu.SemaphoreType.DMA,
        pltpu.SemaphoreType.DMA,
        pltpu.SemaphoreType.DMA,
        pltpu.SemaphoreType.DMA,
    ],
)
def _edge_kernel(g_hbm, src_hbm, dst_hbm, zeros_hbm, out_hbm,
                 acc_sh, src_all, dst0, dst1, dstt, rows0, rows1, rowst,
                 gsem0, gsem1, dsem0, dsem1, tsem):
    cid = lax.axis_index("c")
    sid = lax.axis_index("s")
    wid = cid * NS + sid
    ebase = wid * EPW
    pltpu.sync_copy(src_hbm.at[pl.ds(ebase, EPW)], src_all)
    pltpu.sync_copy(zeros_hbm.at[pl.ds(sid * RPW, RPW)],
                    acc_sh.at[pl.ds(sid * RPW, RPW)])
    plsc.subcore_barrier()

    def _gather(i, rows, sem):
        pltpu.async_copy(g_hbm.at[src_all.at[pl.ds(i * CH, CH)]], rows, sem)

    def _dload(i, dstb, sem):
        pltpu.async_copy(dst_hbm.at[pl.ds(ebase + i * CH, CH)], dstb, sem)

    def _wait(dst_buf, sem, n):
        pltpu.make_async_copy(zeros_hbm.at[pl.ds(0, n)]
                              if len(dst_buf.shape) == 2 else
                              dst_hbm.at[pl.ds(0, n)],
                              dst_buf, sem).wait()

    _gather(0, rows0, gsem0)
    _dload(0, dst0, dsem0)
    pltpu.async_copy(
        g_hbm.at[src_all.at[pl.ds(NCHUNK * CH, TAIL)]], rowst, tsem)
    pltpu.async_copy(dst_hbm.at[pl.ds(ebase + NCHUNK * CH, TAIL)], dstt, tsem)

    def step2(t, carry):
        i0 = 2 * t
        _gather(i0 + 1, rows1, gsem1)
        _dload(i0 + 1, dst1, dsem1)
        _wait(rows0, gsem0, CH)
        _wait(dst0, dsem0, CH)
        pltpu.sync_copy(rows0, acc_sh.at[dst0], add=True)

        @pl.when(t < NCHUNK // 2 - 1)
        def _():
            _gather(i0 + 2, rows0, gsem0)
            _dload(i0 + 2, dst0, dsem0)

        _wait(rows1, gsem1, CH)
        _wait(dst1, dsem1, CH)
        pltpu.sync_copy(rows1, acc_sh.at[dst1], add=True)
        return carry

    lax.fori_loop(0, NCHUNK // 2, step2, 0)
    _wait(rowst, tsem, TAIL)
    _wait(dstt, tsem, TAIL)
    pltpu.sync_copy(rowst, acc_sh.at[dstt], add=True)
    plsc.subcore_barrier()
    pltpu.sync_copy(acc_sh.at[pl.ds(sid * RPW, RPW)],
                    out_hbm.at[cid, pl.ds(sid * RPW, RPW)])


def _t_matmul_body(x_ref, w_ref, h_ref):
    h_ref[...] = jnp.dot(x_ref[...], w_ref[...],
                         preferred_element_type=jnp.float32)


def _t_scale_body(h_ref, q_ref, g_ref, disv_ref):
    deg = q_ref[0, :, 0:1] + q_ref[1, :, 0:1] + 1.0
    dis = lax.rsqrt(deg)
    g_ref[...] = h_ref[...] * dis
    disv_ref[...] = jnp.broadcast_to(dis, (BLK, 8))


def _t_mid_body(p_ref, g_ref, disv_ref, b_ref, w_ref, gout_ref):
    dis = disv_ref[:, 0:1]
    p = p_ref[...]
    y = dis * (p[0] + p[1] + g_ref[...]) + b_ref[...]
    y = jnp.maximum(y, 0.0)
    h = jnp.dot(y, w_ref[...], preferred_element_type=jnp.float32)
    gout_ref[...] = h * dis


def _t_last_body(p_ref, g_ref, disv_ref, b_ref, out_ref):
    dis = disv_ref[:, 0:1]
    p = p_ref[...]
    out_ref[...] = dis * (p[0] + p[1] + g_ref[...]) + b_ref[...]


def _t_split_body(e_ref, s_ref, d_ref):
    s_ref[...] = e_ref[0]
    d_ref[...] = e_ref[1]


_out_e = jax.ShapeDtypeStruct((E,), jnp.int32)
_t_split = pl.pallas_call(
    _t_split_body,
    in_specs=[pl.BlockSpec((2, E), lambda: (0, 0))],
    out_specs=(pl.BlockSpec((E,), lambda: (0,)),
               pl.BlockSpec((E,), lambda: (0,))),
    out_shape=(_out_e, _out_e))

_GRID = (N_PAD // BLK,)
_spec_rows = pl.BlockSpec((BLK, D), lambda i: (i, 0))
_spec_p = pl.BlockSpec((2, BLK, D), lambda i: (0, i, 0))
_spec_q = pl.BlockSpec((2, BLK, D), lambda i: (0, i, 0))
_spec_dis = pl.BlockSpec((BLK, 8), lambda i: (i, 0))
_spec_w = pl.BlockSpec((D, D), lambda i: (0, 0))
_spec_b = pl.BlockSpec((1, D), lambda i: (0, 0))
_out_rows = jax.ShapeDtypeStruct((N_PAD, D), jnp.float32)
_out_dis = jax.ShapeDtypeStruct((N_PAD, 8), jnp.float32)

_t_matmul = pl.pallas_call(
    _t_matmul_body, grid=_GRID,
    in_specs=[_spec_rows, _spec_w],
    out_specs=_spec_rows, out_shape=_out_rows)

_t_scale = pl.pallas_call(
    _t_scale_body, grid=_GRID,
    in_specs=[_spec_rows, _spec_q],
    out_specs=(_spec_rows, _spec_dis), out_shape=(_out_rows, _out_dis))

_t_mid = pl.pallas_call(
    _t_mid_body, grid=_GRID,
    in_specs=[_spec_p, _spec_rows, _spec_dis, _spec_b, _spec_w],
    out_specs=_spec_rows, out_shape=_out_rows)

_t_last = pl.pallas_call(
    _t_last_body, grid=(N // BLKL,),
    in_specs=[pl.BlockSpec((2, BLKL, D), lambda i: (0, i, 0)),
              pl.BlockSpec((BLKL, D), lambda i: (i, 0)),
              pl.BlockSpec((BLKL, 8), lambda i: (i, 0)),
              _spec_b],
    out_specs=pl.BlockSpec((BLKL, D), lambda i: (i, 0)),
    out_shape=jax.ShapeDtypeStruct((N, D), jnp.float32))


def kernel(x, edge_index, W1, b1, W2, b2, W3, b3):
    src_p, dst_p = _t_split(edge_index.astype(jnp.int32))
    x_p = jnp.pad(x, ((0, N_PAD - N), (0, 0)))
    zeros_rows = jnp.zeros((N_PAD, D), jnp.float32)
    ones_rows = jnp.ones((CH, D), jnp.float32)
    b1r = b1.reshape(1, D)
    b2r = b2.reshape(1, D)
    b3r = b3.reshape(1, D)

    q = _deg_kernel(dst_p, ones_rows, zeros_rows)
    h1 = _t_matmul(x_p, W1)
    g1, disv = _t_scale(h1, q)
    p1 = _edge_kernel(g1, src_p, dst_p, zeros_rows)
    g2 = _t_mid(p1, g1, disv, b1r, W2)
    p2 = _edge_kernel(g2, src_p, dst_p, zeros_rows)
    g3 = _t_mid(p2, g2, disv, b2r, W3)
    p3 = _edge_kernel(g3, src_p, dst_p, zeros_rows)
    return _t_last(p3, g3, disv, b3r)
